# Initial kernel scaffold; baseline (speedup 1.0000x reference)
import jax
import jax.numpy as jnp
from jax.experimental import pallas as pl


def kernel(probs, Wn, bn, We, be, eW1, eb1, eW2, eb2, nW1, nb1, nW2, nb2, Wc, bc):
    raise NotImplementedError("write your pallas kernel here")


# speedup vs baseline: 26.4602x; 26.4602x over previous
import functools

import jax
import jax.numpy as jnp
from jax.experimental import pallas as pl

NBIRD = 50000
NCOLOR = 16
H = 64
BB = 1000
NBLK = NBIRD // BB
F32 = jnp.float32


def _layer0_body(probs_ref, wnT_ref, bn_ref, aT_ref, u_ref, base_ref,
                 e2T_ref, eb2_ref, n1aT_ref, n1bT_ref, nb1_ref, n2T_ref,
                 nb2_ref, e_out_ref, x_out_ref, csum_ref):
    p = probs_ref[:]
    xb = jnp.dot(p, wnT_ref[:], preferred_element_type=F32) + bn_ref[:]
    ba = jnp.dot(xb, aT_ref[:], preferred_element_type=F32)
    pre = (ba[:, None, :] + p[:, :, None] * u_ref[:][None, :, :]
           + base_ref[:][None, :, :])
    h = jnp.maximum(pre, 0.0).reshape(BB * NCOLOR, H)
    e_new = jnp.dot(h, e2T_ref[:], preferred_element_type=F32) + eb2_ref[:]
    e_out_ref[:] = e_new.astype(e_out_ref.dtype)
    e3 = e_new.reshape(BB, NCOLOR, H)
    aggr = jnp.sum(e3, axis=1)
    part = jnp.sum(e3, axis=0)
    h2 = jnp.maximum(
        jnp.dot(xb, n1aT_ref[:], preferred_element_type=F32)
        + jnp.dot(aggr, n1bT_ref[:], preferred_element_type=F32)
        + nb1_ref[:], 0.0)
    x_out_ref[:] = jnp.dot(h2, n2T_ref[:], preferred_element_type=F32) + nb2_ref[:]
    pid = pl.program_id(0)

    @pl.when(pid == 0)
    def _():
        csum_ref[:] = part

    @pl.when(pid > 0)
    def _():
        csum_ref[:] = csum_ref[:] + part


def _mid_body(e_ref, x_ref, aT_ref, cT_ref, base_ref, e2T_ref, eb2_ref,
              n1aT_ref, n1bT_ref, nb1_ref, n2T_ref, nb2_ref,
              e_out_ref, x_out_ref, csum_ref):
    x = x_ref[:]
    ba = jnp.dot(x, aT_ref[:], preferred_element_type=F32)
    ec = jnp.dot(e_ref[:].astype(F32), cT_ref[:], preferred_element_type=F32)
    pre = ec.reshape(BB, NCOLOR, H) + ba[:, None, :] + base_ref[:][None, :, :]
    h = jnp.maximum(pre, 0.0).reshape(BB * NCOLOR, H)
    e_new = jnp.dot(h, e2T_ref[:], preferred_element_type=F32) + eb2_ref[:]
    e_out_ref[:] = e_new.astype(e_out_ref.dtype)
    e3 = e_new.reshape(BB, NCOLOR, H)
    aggr = jnp.sum(e3, axis=1)
    part = jnp.sum(e3, axis=0)
    h2 = jnp.maximum(
        jnp.dot(x, n1aT_ref[:], preferred_element_type=F32)
        + jnp.dot(aggr, n1bT_ref[:], preferred_element_type=F32)
        + nb1_ref[:], 0.0)
    x_out_ref[:] = jnp.dot(h2, n2T_ref[:], preferred_element_type=F32) + nb2_ref[:]
    pid = pl.program_id(0)

    @pl.when(pid == 0)
    def _():
        csum_ref[:] = part

    @pl.when(pid > 0)
    def _():
        csum_ref[:] = csum_ref[:] + part


def _last_body(e_ref, x_ref, probs_ref, aT_ref, cT_ref, base_ref, e2T_ref,
               eb2_ref, n1aT_ref, n1bT_ref, nb1_ref, n2T_ref, nb2_ref,
               wcT_ref, bc_ref, out_ref):
    x = x_ref[:]
    ba = jnp.dot(x, aT_ref[:], preferred_element_type=F32)
    ec = jnp.dot(e_ref[:].astype(F32), cT_ref[:], preferred_element_type=F32)
    pre = ec.reshape(BB, NCOLOR, H) + ba[:, None, :] + base_ref[:][None, :, :]
    h = jnp.maximum(pre, 0.0).reshape(BB * NCOLOR, H)
    e_new = jnp.dot(h, e2T_ref[:], preferred_element_type=F32) + eb2_ref[:]
    e3 = e_new.reshape(BB, NCOLOR, H)
    aggr = jnp.sum(e3, axis=1)
    h2 = jnp.maximum(
        jnp.dot(x, n1aT_ref[:], preferred_element_type=F32)
        + jnp.dot(aggr, n1bT_ref[:], preferred_element_type=F32)
        + nb1_ref[:], 0.0)
    xn = jnp.dot(h2, n2T_ref[:], preferred_element_type=F32) + nb2_ref[:]
    scores = jnp.dot(xn, wcT_ref[:], preferred_element_type=F32) + bc_ref[:]
    out_ref[:] = scores * probs_ref[:]


def _full(shape):
    return pl.BlockSpec(shape, lambda i: tuple(0 for _ in shape))


_EDGE_DT = jnp.float32


def kernel(probs, Wn, bn, We, be, eW1, eb1, eW2, eb2, nW1, nb1, nW2, nb2,
           Wc, bc):
    f = lambda a: a.astype(F32)
    probs = f(probs)
    wnT = f(Wn).T
    x_color = wnT + f(bn)[None, :]
    A = [f(eW1[l][:, :H]).T for l in range(3)]
    Bm = [f(eW1[l][:, H:2 * H]).T for l in range(3)]
    Cm = [f(eW1[l][:, 2 * H:]).T for l in range(3)]
    E2 = [f(eW2[l]).T for l in range(3)]
    N1a = [f(nW1[l][:, :H]).T for l in range(3)]
    N1b = [f(nW1[l][:, H:]).T for l in range(3)]
    N2 = [f(nW2[l]).T for l in range(3)]
    eb1_ = [f(eb1[l])[None, :] for l in range(3)]
    eb2_ = [f(eb2[l])[None, :] for l in range(3)]
    nb1_ = [f(nb1[l])[None, :] for l in range(3)]
    nb2_ = [f(nb2[l])[None, :] for l in range(3)]
    bn_r = f(bn)[None, :]
    u0 = (f(We)[:, 0] @ Cm[0])[None, :]
    v0 = (f(be) @ Cm[0])[None, :]

    def edge_base(l, xc):
        b = xc @ Bm[l] + eb1_[l]
        if l == 0:
            b = b + v0
        return b

    def color_update(l, xc, aggr_c):
        h2 = jnp.maximum(xc @ N1a[l] + aggr_c @ N1b[l] + nb1_[l], 0.0)
        return h2 @ N2[l] + nb2_[l]

    wspecs = [_full((H, H)), _full((H, H)), _full((NCOLOR, H)),
              _full((H, H)), _full((1, H)), _full((H, H)), _full((H, H)),
              _full((1, H)), _full((H, H)), _full((1, H))]
    e_spec = pl.BlockSpec((BB * NCOLOR, H), lambda i: (i, 0))
    x_spec = pl.BlockSpec((BB, H), lambda i: (i, 0))
    p_spec = pl.BlockSpec((BB, NCOLOR), lambda i: (i, 0))
    csum_spec = pl.BlockSpec((NCOLOR, H), lambda i: (0, 0))

    e1, x1, csum = pl.pallas_call(
        _layer0_body,
        grid=(NBLK,),
        in_specs=[p_spec, _full((NCOLOR, H)), _full((1, H)), _full((H, H)),
                  _full((1, H)), _full((NCOLOR, H)), _full((H, H)),
                  _full((1, H)), _full((H, H)), _full((H, H)), _full((1, H)),
                  _full((H, H)), _full((1, H))],
        out_specs=[e_spec, x_spec, csum_spec],
        out_shape=[
            jax.ShapeDtypeStruct((NBIRD * NCOLOR, H), _EDGE_DT),
            jax.ShapeDtypeStruct((NBIRD, H), F32),
            jax.ShapeDtypeStruct((NCOLOR, H), F32),
        ],
    )(probs, wnT, bn_r, A[0], u0, edge_base(0, x_color), E2[0], eb2_[0],
      N1a[0], N1b[0], nb1_[0], N2[0], nb2_[0])
    x_color = color_update(0, x_color, csum)

    e2, x2, csum = pl.pallas_call(
        _mid_body,
        grid=(NBLK,),
        in_specs=[e_spec, x_spec] + wspecs,
        out_specs=[e_spec, x_spec, csum_spec],
        out_shape=[
            jax.ShapeDtypeStruct((NBIRD * NCOLOR, H), _EDGE_DT),
            jax.ShapeDtypeStruct((NBIRD, H), F32),
            jax.ShapeDtypeStruct((NCOLOR, H), F32),
        ],
    )(e1, x1, A[1], Cm[1], edge_base(1, x_color), E2[1], eb2_[1],
      N1a[1], N1b[1], nb1_[1], N2[1], nb2_[1])
    x_color = color_update(1, x_color, csum)

    out = pl.pallas_call(
        _last_body,
        grid=(NBLK,),
        in_specs=[e_spec, x_spec, p_spec] + wspecs + [_full((H, NCOLOR)),
                                                      _full((1, NCOLOR))],
        out_specs=p_spec,
        out_shape=jax.ShapeDtypeStruct((NBIRD, NCOLOR), F32),
    )(e2, x2, probs, A[2], Cm[2], edge_base(2, x_color), E2[2], eb2_[2],
      N1a[2], N1b[2], nb1_[2], N2[2], nb2_[2], f(Wc).T, f(bc)[None, :])
    return out
